# Initial kernel scaffold; baseline (speedup 1.0000x reference)
import jax
import jax.numpy as jnp
from jax.experimental import pallas as pl


def kernel(user, movie, actor, country, movie_type, user_table, movie_table, actor_table, country_table, type_table, W1, b1, W2, b2, W3, b3):
    raise NotImplementedError("write your pallas kernel here")


# speedup vs baseline: 4.4557x; 4.4557x over previous
import functools

import jax
import jax.numpy as jnp
from jax import lax
from jax.experimental import pallas as pl
from jax.experimental.pallas import tpu as pltpu
from jax.experimental.pallas import tpu_sc as plsc

B = 16384
D = 32
NC, NS = 2, 16
NW = NC * NS
BPW = B // NW
N_ACTOR, N_COUNTRY, N_TYPE = 20, 4, 8


def _sc_gather_body(user_hbm, movie_hbm, actor_hbm, country_hbm, type_hbm,
                    ut_hbm, mt_hbm, at_hbm, ct_hbm, tt_hbm,
                    u_out, m_out, a_out, c_out, t_out,
                    uidx_v, midx_v, aidx_v, cidx_v, tidx_v, acc_v, sem):
    wid = lax.axis_index("s") * NC + lax.axis_index("c")
    base = wid * BPW

    pltpu.sync_copy(user_hbm.at[pl.ds(base, BPW)], uidx_v)
    pltpu.sync_copy(movie_hbm.at[pl.ds(base, BPW)], midx_v)
    pltpu.sync_copy(actor_hbm.at[:, pl.ds(base, BPW)], aidx_v)
    pltpu.sync_copy(country_hbm.at[:, pl.ds(base, BPW)], cidx_v)
    pltpu.sync_copy(type_hbm.at[:, pl.ds(base, BPW)], tidx_v)

    pltpu.async_copy(ut_hbm.at[uidx_v], acc_v, sem).wait()
    pltpu.sync_copy(acc_v, u_out.at[pl.ds(base, BPW)])
    pltpu.async_copy(mt_hbm.at[midx_v], acc_v, sem).wait()
    pltpu.sync_copy(acc_v, m_out.at[pl.ds(base, BPW)])

    def pooled(idx_v, n, table, out):
        pltpu.async_copy(table.at[idx_v.at[0]], acc_v, sem).wait()

        def body(j, carry):
            pltpu.async_copy(table.at[idx_v.at[j]], acc_v, sem, add=True).wait()
            return carry

        lax.fori_loop(1, n, body, 0)
        pltpu.sync_copy(acc_v, out.at[pl.ds(base, BPW)])

    pooled(aidx_v, N_ACTOR, at_hbm, a_out)
    pooled(cidx_v, N_COUNTRY, ct_hbm, c_out)
    pooled(tidx_v, N_TYPE, tt_hbm, t_out)


@functools.partial(jax.jit, static_argnames=())
def _sc_gather(user, movie, actor_t, country_t, type_t,
               user_table, movie_table, actor_table, country_table, type_table):
    emb = jax.ShapeDtypeStruct((B, D), jnp.float32)
    run = pl.kernel(
        _sc_gather_body,
        out_type=(emb, emb, emb, emb, emb),
        mesh=plsc.VectorSubcoreMesh(core_axis_name="c", subcore_axis_name="s",
                                    num_cores=NC, num_subcores=NS),
        scratch_types=[
            pltpu.VMEM((BPW,), jnp.int32),
            pltpu.VMEM((BPW,), jnp.int32),
            pltpu.VMEM((N_ACTOR, BPW), jnp.int32),
            pltpu.VMEM((N_COUNTRY, BPW), jnp.int32),
            pltpu.VMEM((N_TYPE, BPW), jnp.int32),
            pltpu.VMEM((BPW, D), jnp.float32),
            pltpu.SemaphoreType.DMA,
        ],
        compiler_params=pltpu.CompilerParams(use_tc_tiling_on_sc=False),
    )
    return run(user, movie, actor_t, country_t, type_t,
               user_table, movie_table, actor_table, country_table, type_table)


def _mlp_body(u, m, a, c, t, w1, b1, w2, b2, w3, b3, out):
    f32 = jnp.float32
    h = (jnp.dot(u[...], w1[0:D, :], preferred_element_type=f32)
         + jnp.dot(m[...], w1[D:2 * D, :], preferred_element_type=f32)
         + jnp.dot(a[...] * (1.0 / N_ACTOR), w1[2 * D:3 * D, :], preferred_element_type=f32)
         + jnp.dot(c[...] * (1.0 / N_COUNTRY), w1[3 * D:4 * D, :], preferred_element_type=f32)
         + jnp.dot(t[...] * (1.0 / N_TYPE), w1[4 * D:5 * D, :], preferred_element_type=f32)
         + b1[...])
    h = jnp.maximum(h, 0.0)
    h2 = jnp.maximum(jnp.dot(h, w2[...], preferred_element_type=f32) + b2[...], 0.0)
    out[...] = jnp.dot(h2, w3[...], preferred_element_type=f32) + b3[...]


def _mlp(u, m, a, c, t, W1, b1, W2, b2, W3, b3):
    BM = 2048
    grid = (B // BM,)
    emb_spec = pl.BlockSpec((BM, D), lambda i: (i, 0))
    full = lambda s: pl.BlockSpec(s, lambda i: tuple(0 for _ in s))
    return pl.pallas_call(
        _mlp_body,
        grid=grid,
        in_specs=[emb_spec, emb_spec, emb_spec, emb_spec, emb_spec,
                  full((5 * D, 64)), full((64,)), full((64, 32)), full((32,)),
                  full((32, 1)), full((1,))],
        out_specs=pl.BlockSpec((BM, 1), lambda i: (i, 0)),
        out_shape=jax.ShapeDtypeStruct((B, 1), jnp.float32),
    )(u, m, a, c, t, W1, b1, W2, b2, W3, b3)


def kernel(user, movie, actor, country, movie_type,
           user_table, movie_table, actor_table, country_table, type_table,
           W1, b1, W2, b2, W3, b3):
    user = user.astype(jnp.int32)
    actor_t = actor.T
    country_t = country.T
    type_t = movie_type.T
    u, m, a, c, t = _sc_gather(user, movie, actor_t, country_t, type_t,
                               user_table, movie_table, actor_table,
                               country_table, type_table)
    y = _mlp(u, m, a, c, t, W1, b1, W2, b2, W3, b3)
    return jnp.squeeze(y, axis=-1)
